# Initial kernel scaffold; baseline (speedup 1.0000x reference)
import jax
import jax.numpy as jnp
from jax.experimental import pallas as pl


def kernel(x, edge_index, W1, b1, W2, b2, W3, b3, W4, b4):
    raise NotImplementedError("write your pallas kernel here")


# speedup vs baseline: 3.1816x; 3.1816x over previous
import functools

import jax
import jax.numpy as jnp
from jax import lax
from jax.experimental import pallas as pl
from jax.experimental.pallas import tpu as pltpu
from jax.experimental.pallas import tpu_sc as plsc

NC = 2
NS = 16
LANES = 16

EB = 80


def _make_gather_kernel(n_edges, d):
  e_per_w = n_edges // (NC * NS)
  n_chunks = e_per_w // EB
  mesh = plsc.VectorSubcoreMesh(core_axis_name="c", subcore_axis_name="s")

  @functools.partial(
      pl.kernel,
      out_type=jax.ShapeDtypeStruct((n_edges, d), jnp.float32),
      mesh=mesh,
      scratch_types=[
          pltpu.VMEM((EB,), jnp.int32),
          pltpu.VMEM((EB, d), jnp.float32),
          pltpu.VMEM((EB, d), jnp.float32),
          pltpu.SemaphoreType.DMA,
          pltpu.SemaphoreType.DMA,
      ],
  )
  def gather_kernel(g_hbm, src_hbm, out_hbm, idx_v, rows_a, rows_b, sem_a, sem_b):
    c = lax.axis_index("c")
    s = lax.axis_index("s")
    wid = s * NC + c
    base0 = wid * e_per_w

    def body(i, _):
      base = base0 + i * EB
      pltpu.sync_copy(src_hbm.at[pl.ds(base, EB)], idx_v)
      pltpu.async_copy(g_hbm.at[idx_v], rows_a, sem_a).wait()
      pltpu.sync_copy(rows_a, out_hbm.at[pl.ds(base, EB)])
      return 0

    lax.fori_loop(0, n_chunks, body, 0)

  return gather_kernel


_ROWS = 400


def _tc_first_body(x_ref, w_ref, dinv_ref, out_ref):
  dinv = dinv_ref[:, 0:1]
  t = jnp.dot(x_ref[...], w_ref[...], preferred_element_type=jnp.float32)
  out_ref[...] = t * dinv


def _tc_mid_body(p_ref, g_ref, dinv_ref, b_ref, w_ref, out_ref):
  dinv = dinv_ref[:, 0:1]
  h = jnp.maximum((p_ref[...] + g_ref[...]) * dinv + b_ref[...], 0.0)
  t = jnp.dot(h, w_ref[...], preferred_element_type=jnp.float32)
  out_ref[...] = t * dinv


def _tc_final_body(p_ref, g_ref, dinv_ref, b_ref, out_ref):
  dinv = dinv_ref[:, 0:1]
  d_out = out_ref.shape[-1]
  s = (p_ref[...] + g_ref[...])[:, :d_out]
  out_ref[...] = s * dinv + b_ref[...]


def _tc_first(x, w, dinv16):
  n, d_in = x.shape
  d_out = w.shape[1]
  return pl.pallas_call(
      _tc_first_body,
      grid=(n // _ROWS,),
      in_specs=[
          pl.BlockSpec((_ROWS, d_in), lambda i: (i, 0)),
          pl.BlockSpec((d_in, d_out), lambda i: (0, 0)),
          pl.BlockSpec((_ROWS, LANES), lambda i: (i, 0)),
      ],
      out_specs=pl.BlockSpec((_ROWS, d_out), lambda i: (i, 0)),
      out_shape=jax.ShapeDtypeStruct((n, d_out), jnp.float32),
  )(x, w, dinv16)


def _tc_mid(p, g, dinv16, b, w):
  n, d_in = g.shape
  d_out = w.shape[1]
  return pl.pallas_call(
      _tc_mid_body,
      grid=(n // _ROWS,),
      in_specs=[
          pl.BlockSpec((_ROWS, d_in), lambda i: (i, 0)),
          pl.BlockSpec((_ROWS, d_in), lambda i: (i, 0)),
          pl.BlockSpec((_ROWS, LANES), lambda i: (i, 0)),
          pl.BlockSpec((1, d_in), lambda i: (0, 0)),
          pl.BlockSpec((d_in, d_out), lambda i: (0, 0)),
      ],
      out_specs=pl.BlockSpec((_ROWS, d_out), lambda i: (i, 0)),
      out_shape=jax.ShapeDtypeStruct((n, d_out), jnp.float32),
  )(p, g, dinv16, b, w)


def _tc_final(p, g, dinv16, b):
  n, d_in = g.shape
  d_out = b.shape[-1]
  return pl.pallas_call(
      _tc_final_body,
      grid=(n // _ROWS,),
      in_specs=[
          pl.BlockSpec((_ROWS, d_in), lambda i: (i, 0)),
          pl.BlockSpec((_ROWS, d_in), lambda i: (i, 0)),
          pl.BlockSpec((_ROWS, LANES), lambda i: (i, 0)),
          pl.BlockSpec((1, d_out), lambda i: (0, 0)),
      ],
      out_specs=pl.BlockSpec((_ROWS, d_out), lambda i: (i, 0)),
      out_shape=jax.ShapeDtypeStruct((n, d_out), jnp.float32),
  )(p, g, dinv16, b)


def kernel(x, edge_index, W1, b1, W2, b2, W3, b3, W4, b4):
  n, _ = x.shape
  n_edges = edge_index.shape[1]
  src = edge_index[0]
  dst = edge_index[1]

  gath = _make_gather_kernel(n_edges, 128)

  deg = jax.ops.segment_sum(jnp.ones((n_edges,), jnp.float32), dst,
                            num_segments=n) + 1.0
  dinv = lax.rsqrt(deg)
  dinv16 = jnp.broadcast_to(dinv[:, None], (n, LANES))

  w4p = jnp.pad(W4, ((0, 0), (0, 128 - W4.shape[1])))

  def propagate(g):
    m = gath(g, src)
    return jax.ops.segment_sum(m, dst, num_segments=n)

  g1 = _tc_first(x, W1, dinv16)
  p = propagate(g1)
  g2 = _tc_mid(p, g1, dinv16, b1.reshape(1, -1), W2)
  p = propagate(g2)
  g3 = _tc_mid(p, g2, dinv16, b2.reshape(1, -1), W3)
  p = propagate(g3)
  g4 = _tc_mid(p, g3, dinv16, b3.reshape(1, -1), w4p)
  p = propagate(g4)
  return _tc_final(p, g4, dinv16, b4.reshape(1, -1))


# speedup vs baseline: 3.4893x; 1.0967x over previous
import functools

import jax
import jax.numpy as jnp
from jax import lax
from jax.experimental import pallas as pl
from jax.experimental.pallas import tpu as pltpu
from jax.experimental.pallas import tpu_sc as plsc

NC = 2
NS = 16
LANES = 16

EB = 80


def _make_gather_kernel(n_edges, d):
  e_per_w = n_edges // (NC * NS)
  n_chunks = e_per_w // EB
  mesh = plsc.VectorSubcoreMesh(core_axis_name="c", subcore_axis_name="s")

  @functools.partial(
      pl.kernel,
      out_type=jax.ShapeDtypeStruct((n_edges, d), jnp.float32),
      mesh=mesh,
      scratch_types=[
          pltpu.VMEM((EB,), jnp.int32),
          pltpu.VMEM((EB,), jnp.int32),
          pltpu.VMEM((EB, d), jnp.float32),
          pltpu.VMEM((EB, d), jnp.float32),
          pltpu.SemaphoreType.DMA,
          pltpu.SemaphoreType.DMA,
      ],
  )
  def gather_kernel(g_hbm, src_hbm, out_hbm, idx_v, idx_b, rows_a, rows_b,
                    sem_a, sem_b):
    c = lax.axis_index("c")
    s = lax.axis_index("s")
    wid = s * NC + c
    base0 = wid * e_per_w

    def body(i, _):
      base = base0 + 2 * i * EB

      pltpu.sync_copy(src_hbm.at[pl.ds(base, EB)], idx_v)
      pltpu.async_copy(g_hbm.at[idx_v], rows_a, sem_a)
      pltpu.sync_copy(src_hbm.at[pl.ds(base + EB, EB)], idx_b)
      pltpu.async_copy(g_hbm.at[idx_b], rows_b, sem_b)

      pltpu.make_async_copy(g_hbm.at[idx_v], rows_a, sem_a).wait()
      pltpu.sync_copy(rows_a, out_hbm.at[pl.ds(base, EB)])
      pltpu.make_async_copy(g_hbm.at[idx_b], rows_b, sem_b).wait()
      pltpu.sync_copy(rows_b, out_hbm.at[pl.ds(base + EB, EB)])
      return 0

    lax.fori_loop(0, n_chunks // 2, body, 0)

    if n_chunks % 2:
      base = base0 + (n_chunks - 1) * EB
      pltpu.sync_copy(src_hbm.at[pl.ds(base, EB)], idx_v)
      pltpu.async_copy(g_hbm.at[idx_v], rows_a, sem_a).wait()
      pltpu.sync_copy(rows_a, out_hbm.at[pl.ds(base, EB)])

  return gather_kernel


_ROWS = 400


def _tc_first_body(x_ref, w_ref, dinv_ref, out_ref):
  dinv = dinv_ref[:, 0:1]
  t = jnp.dot(x_ref[...], w_ref[...], preferred_element_type=jnp.float32)
  out_ref[...] = t * dinv


def _tc_mid_body(p_ref, g_ref, dinv_ref, b_ref, w_ref, out_ref):
  dinv = dinv_ref[:, 0:1]
  h = jnp.maximum((p_ref[...] + g_ref[...]) * dinv + b_ref[...], 0.0)
  t = jnp.dot(h, w_ref[...], preferred_element_type=jnp.float32)
  out_ref[...] = t * dinv


def _tc_final_body(p_ref, g_ref, dinv_ref, b_ref, out_ref):
  dinv = dinv_ref[:, 0:1]
  d_out = out_ref.shape[-1]
  s = p_ref[...] + g_ref[...][:, :d_out]
  out_ref[...] = s * dinv + b_ref[...]


def _tc_first(x, w, dinv16):
  n, d_in = x.shape
  d_out = w.shape[1]
  return pl.pallas_call(
      _tc_first_body,
      grid=(n // _ROWS,),
      in_specs=[
          pl.BlockSpec((_ROWS, d_in), lambda i: (i, 0)),
          pl.BlockSpec((d_in, d_out), lambda i: (0, 0)),
          pl.BlockSpec((_ROWS, LANES), lambda i: (i, 0)),
      ],
      out_specs=pl.BlockSpec((_ROWS, d_out), lambda i: (i, 0)),
      out_shape=jax.ShapeDtypeStruct((n, d_out), jnp.float32),
  )(x, w, dinv16)


def _tc_mid(p, g, dinv16, b, w):
  n, d_in = g.shape
  d_out = w.shape[1]
  return pl.pallas_call(
      _tc_mid_body,
      grid=(n // _ROWS,),
      in_specs=[
          pl.BlockSpec((_ROWS, d_in), lambda i: (i, 0)),
          pl.BlockSpec((_ROWS, d_in), lambda i: (i, 0)),
          pl.BlockSpec((_ROWS, LANES), lambda i: (i, 0)),
          pl.BlockSpec((1, d_in), lambda i: (0, 0)),
          pl.BlockSpec((d_in, d_out), lambda i: (0, 0)),
      ],
      out_specs=pl.BlockSpec((_ROWS, d_out), lambda i: (i, 0)),
      out_shape=jax.ShapeDtypeStruct((n, d_out), jnp.float32),
  )(p, g, dinv16, b, w)


def _tc_final(p, g, dinv16, b):
  n, d_in = g.shape
  d_out = b.shape[-1]
  return pl.pallas_call(
      _tc_final_body,
      grid=(n // _ROWS,),
      in_specs=[
          pl.BlockSpec((_ROWS, d_out), lambda i: (i, 0)),
          pl.BlockSpec((_ROWS, d_in), lambda i: (i, 0)),
          pl.BlockSpec((_ROWS, LANES), lambda i: (i, 0)),
          pl.BlockSpec((1, d_out), lambda i: (0, 0)),
      ],
      out_specs=pl.BlockSpec((_ROWS, d_out), lambda i: (i, 0)),
      out_shape=jax.ShapeDtypeStruct((n, d_out), jnp.float32),
  )(p, g, dinv16, b)


def kernel(x, edge_index, W1, b1, W2, b2, W3, b3, W4, b4):
  n, _ = x.shape
  n_edges = edge_index.shape[1]
  src = edge_index[0]
  dst = edge_index[1]

  gath = _make_gather_kernel(n_edges, 128)

  deg = jax.ops.segment_sum(jnp.ones((n_edges,), jnp.float32), dst,
                            num_segments=n) + 1.0
  dinv = lax.rsqrt(deg)
  dinv16 = jnp.broadcast_to(dinv[:, None], (n, LANES))

  w4p = jnp.pad(W4, ((0, 0), (0, 128 - W4.shape[1])))

  def propagate(g):
    m = gath(g, src)
    return jax.ops.segment_sum(m, dst, num_segments=n)

  g1 = _tc_first(x, W1, dinv16)
  p = propagate(g1)
  g2 = _tc_mid(p, g1, dinv16, b1.reshape(1, -1), W2)
  p = propagate(g2)
  g3 = _tc_mid(p, g2, dinv16, b2.reshape(1, -1), W3)
  p = propagate(g3)
  g4 = _tc_mid(p, g3, dinv16, b3.reshape(1, -1), w4p)
  m4 = gath(g4, src)
  p = jax.ops.segment_sum(m4[:, :64], dst, num_segments=n)
  return _tc_final(p, g4, dinv16, b4.reshape(1, -1))
